# baseline (speedup 1.0000x reference)
import functools

import jax
import jax.numpy as jnp
from jax import lax
from jax.experimental import pallas as pl
from jax.experimental.pallas import tpu as pltpu
from jax.experimental.pallas import tpu_sc as plsc

N = 10000
E = 320000
D = 128
B = 100
K = 50

NW = 32
NPAD = 10240
R = NPAD // NW

EW = E // NW
CG = 200
NCHG = EW // CG
NPAIR = NCHG // 2

CS = 8000
NCHS = E // CS
GB = 64
CSQ = CS // 4
QCAP = CSQ + GB

SEL = B * K
SELPAD = 5120
SELW = SELPAD // NW

_f32 = jnp.float32
_i32 = jnp.int32

_mesh = plsc.VectorSubcoreMesh(core_axis_name="c", subcore_axis_name="s")


def _wid():
    return lax.axis_index("s") * 2 + lax.axis_index("c")


def _iota16():
    return lax.broadcasted_iota(_i32, (16,), 0)


def _dense_uv_body(acc_ref, res_ref, g_ref, bl_ref, w1_ref, b1_ref,
                   xcur_ref, u_ref, v_ref):
    xc = acc_ref[...] + res_ref[...]
    xcur_ref[...] = xc
    mu = jnp.mean(xc, axis=-1, keepdims=True)
    var = jnp.mean((xc - mu) ** 2, axis=-1, keepdims=True)
    h = (xc - mu) * lax.rsqrt(var + 1e-5) * g_ref[...] + bl_ref[...]
    w1 = w1_ref[...]
    wv = w1[D:]
    wu = w1[:D] - wv
    u_ref[...] = jnp.dot(h, wu, preferred_element_type=_f32) + b1_ref[...]
    v_ref[...] = jnp.dot(h, wv, preferred_element_type=_f32)


def _dense_uv(acc, res, g, bl, w1, b1):
    nb = 1280
    grid = NPAD // nb
    row = pl.BlockSpec((nb, D), lambda i: (i, 0))
    full = lambda s: pl.BlockSpec(s, lambda i: (0, 0))
    return pl.pallas_call(
        _dense_uv_body,
        grid=(grid,),
        in_specs=[row, row, full((1, D)), full((1, D)), full((2 * D, D)),
                  full((1, D))],
        out_specs=[row, row, row],
        out_shape=[jax.ShapeDtypeStruct((NPAD, D), _f32)] * 3,
    )(acc, res, g.reshape(1, D), bl.reshape(1, D), w1, b1.reshape(1, D))


@functools.partial(
    pl.kernel,
    out_type=jax.ShapeDtypeStruct((E, D), _f32),
    mesh=_mesh,
    compiler_params=pltpu.CompilerParams(needs_layout_passes=False),
    scratch_types=[
        pltpu.VMEM((EW,), _i32),
        pltpu.VMEM((EW,), _i32),
        pltpu.VMEM((CG, D), _f32),
        pltpu.VMEM((CG, D), _f32),
        pltpu.VMEM((CG, D), _f32),
        pltpu.VMEM((CG, D), _f32),
        pltpu.SemaphoreType.DMA,
        pltpu.SemaphoreType.DMA,
        pltpu.SemaphoreType.DMA,
        pltpu.SemaphoreType.DMA,
        pltpu.SemaphoreType.DMA,
        pltpu.SemaphoreType.DMA,
    ],
)
def _sc_gather_pre(u_hbm, v_hbm, dst_hbm, src_hbm, pre_hbm,
                   dsti, srci, bufua, bufva, bufub, bufvb,
                   sua, sva, subb, svb, soa, sob):
    ebase = _wid() * EW
    pltpu.sync_copy(dst_hbm.at[pl.ds(ebase, EW)], dsti)
    pltpu.sync_copy(src_hbm.at[pl.ds(ebase, EW)], srci)

    def fire(c, bufu, bufv, su, sv):
        pltpu.async_copy(u_hbm.at[dsti.at[pl.ds(c * CG, CG)]], bufu, su)
        pltpu.async_copy(v_hbm.at[srci.at[pl.ds(c * CG, CG)]], bufv, sv)

    def wait_g(c, bufu, bufv, su, sv):
        pltpu.make_async_copy(u_hbm.at[dsti.at[pl.ds(c * CG, CG)]],
                              bufu, su).wait()
        pltpu.make_async_copy(v_hbm.at[srci.at[pl.ds(c * CG, CG)]],
                              bufv, sv).wait()

    def compute(bufu, bufv):
        @plsc.parallel_loop(0, CG, step=1, unroll=8)
        def _row(r):
            for j in range(8):
                sl = pl.ds(16 * j, 16)
                bufu[r, sl] = bufu[r, sl] + bufv[r, sl]

    def out_start(c, bufu, so):
        pltpu.async_copy(bufu, pre_hbm.at[pl.ds(ebase + c * CG, CG)], so)

    def out_wait(c, bufu, so):
        pltpu.make_async_copy(bufu, pre_hbm.at[pl.ds(ebase + c * CG, CG)],
                              so).wait()

    fire(0, bufua, bufva, sua, sva)

    def pair(t, carry):
        ca = 2 * t
        cb = ca + 1

        @pl.when(t > 0)
        def _():
            out_wait(cb - 2, bufub, sob)

        fire(cb, bufub, bufvb, subb, svb)
        wait_g(ca, bufua, bufva, sua, sva)
        compute(bufua, bufva)
        out_start(ca, bufua, soa)
        wait_g(cb, bufub, bufvb, subb, svb)
        compute(bufub, bufvb)
        out_start(cb, bufub, sob)

        @pl.when(t < NPAIR - 1)
        def _():
            out_wait(ca, bufua, soa)
            fire(ca + 2, bufua, bufva, sua, sva)

        return carry

    lax.fori_loop(0, NPAIR, pair, 0)
    out_wait(2 * NPAIR - 2, bufua, soa)
    out_wait(2 * NPAIR - 1, bufub, sob)


def _edge_mlp_body(pre_ref, w2_ref, b2_ref, m_ref):
    m_ref[...] = jnp.dot(jnp.maximum(pre_ref[...], 0.0), w2_ref[...],
                         preferred_element_type=_f32) + b2_ref[...]


def _edge_mlp(pre, w2, b2):
    eb = 3200
    grid = E // eb
    row = pl.BlockSpec((eb, D), lambda i: (i, 0))
    full = lambda s: pl.BlockSpec(s, lambda i: (0, 0))
    return pl.pallas_call(
        _edge_mlp_body,
        grid=(grid,),
        in_specs=[row, full((D, D)), full((1, D))],
        out_specs=row,
        out_shape=jax.ShapeDtypeStruct((E, D), _f32),
    )(pre, w2, b2.reshape(1, D))


@functools.partial(
    pl.kernel,
    out_type=jax.ShapeDtypeStruct((NPAD, D), _f32),
    mesh=_mesh,
    compiler_params=pltpu.CompilerParams(needs_layout_passes=False),
    scratch_types=[
        pltpu.VMEM((CS,), _i32),
        pltpu.VMEM((4 * QCAP,), _i32),
        pltpu.VMEM((4 * QCAP,), _i32),
        pltpu.VMEM((R + 1, D), _f32),
        pltpu.VMEM((R + 1, D), _f32),
        pltpu.VMEM((GB, D), _f32),
        pltpu.VMEM((GB, D), _f32),
        pltpu.SemaphoreType.DMA,
        pltpu.SemaphoreType.DMA,
    ],
)
def _sc_scatter_max(m_hbm, dst_hbm, zacc_hbm, acc_hbm,
                    dstv, mids, mdl, acca, accb, mra, mrb, sma, smb):
    wid = _wid()
    lo = wid * R
    iota = _iota16()
    pltpu.sync_copy(zacc_hbm, acca)
    pltpu.sync_copy(zacc_hbm, accb)
    sent_base = wid * GB

    def chunk(c, carry):
        cbase = c * CS
        pltpu.sync_copy(dst_hbm.at[pl.ds(cbase, CS)], dstv)

        def scan(g, cnts):
            new = []
            for q in range(4):
                cq = cnts[q]
                off = q * CSQ + 16 * g
                dv = plsc.load_gather(dstv, [iota + off])
                dl = dv - lo
                msk = (dl >= 0) & (dl < R)
                plsc.store_compressed(mids.at[pl.ds(q * QCAP + cq, 16)],
                                      iota + (cbase + off), mask=msk)
                plsc.store_compressed(mdl.at[pl.ds(q * QCAP + cq, 16)],
                                      dl, mask=msk)
                pop = plsc.all_reduce_population_count(msk)
                new.append(cq + pop[0])
            return tuple(new)

        cnts = lax.fori_loop(0, CSQ // 16, scan, (0, 0, 0, 0))
        for q in range(4):
            for k in range(GB // 16):
                plsc.store_scatter(mids,
                                   [(q * QCAP + 16 * k) + cnts[q] + iota],
                                   (sent_base + 16 * k) + iota)
                plsc.store_scatter(mdl,
                                   [(q * QCAP + 16 * k) + cnts[q] + iota],
                                   jnp.full((16,), R, _i32))

        def fire(qb, bi, mr, sem):
            pltpu.async_copy(m_hbm.at[mids.at[pl.ds(qb + bi * GB, GB)]],
                             mr, sem)

        def wait(qb, bi, mr, sem):
            pltpu.make_async_copy(m_hbm.at[mids.at[pl.ds(qb + bi * GB, GB)]],
                                  mr, sem).wait()

        def rmw(qb, bi, mr):
            def sub(sg, carry3):
                for i in range(16):
                    p = qb + bi * GB + sg * 16 + i
                    dlv = plsc.load_gather(mdl, [jnp.full((16,), p, _i32)])
                    accx = acca if i % 2 == 0 else accb
                    row = sg * 16 + i
                    mvs = [mr[row, pl.ds(16 * j, 16)] for j in range(8)]
                    olds = [plsc.load_gather(accx, [dlv, iota + 16 * j])
                            for j in range(8)]
                    for j in range(8):
                        plsc.store_scatter(accx, [dlv, iota + 16 * j],
                                           jnp.maximum(olds[j], mvs[j]))
                return carry3

            lax.fori_loop(0, GB // 16, sub, 0)

        for q in range(4):
            qb = q * QCAP
            numb = (cnts[q] + (GB - 1)) // GB

            @pl.when(numb > 0)
            def _(qb=qb):
                fire(qb, 0, mra, sma)

            def batch(bi, carry2, qb=qb, numb=numb):
                even = (bi % 2) == 0
                nxt = bi + 1

                @pl.when((nxt < numb) & even)
                def _():
                    fire(qb, nxt, mrb, smb)

                @pl.when((nxt < numb) & jnp.logical_not(even))
                def _():
                    fire(qb, nxt, mra, sma)

                @pl.when(even)
                def _():
                    wait(qb, bi, mra, sma)
                    rmw(qb, bi, mra)

                @pl.when(jnp.logical_not(even))
                def _():
                    wait(qb, bi, mrb, smb)
                    rmw(qb, bi, mrb)

                return carry2

            lax.fori_loop(0, numb, batch, 0)
        return carry

    lax.fori_loop(0, NCHS, chunk, 0)

    def mrg(r, carry):
        rv = jnp.full((16,), r, _i32)
        for j in range(8):
            cv = iota + 16 * j
            a = plsc.load_gather(acca, [rv, cv])
            b = plsc.load_gather(accb, [rv, cv])
            plsc.store_scatter(acca, [rv, cv], jnp.maximum(a, b))
        return carry

    lax.fori_loop(0, R, mrg, 0)
    pltpu.sync_copy(acca.at[pl.ds(0, R)], acc_hbm.at[pl.ds(lo, R)])


@functools.partial(
    pl.kernel,
    out_type=jax.ShapeDtypeStruct((SELPAD, D), _f32),
    mesh=_mesh,
    compiler_params=pltpu.CompilerParams(needs_layout_passes=False),
    scratch_types=[
        pltpu.VMEM((SELW,), _i32),
        pltpu.VMEM((SELW,), _i32),
        pltpu.VMEM((SELW, D), _f32),
        pltpu.VMEM((SELW, D), _f32),
        pltpu.SemaphoreType.DMA,
        pltpu.SemaphoreType.DMA,
    ],
)
def _sc_gather_sel(acc_hbm, xc_hbm, sel_hbm, out_hbm,
                   selv, gidv, rowa, rowb, s1, s2):
    base = _wid() * SELW
    iota = _iota16()
    pltpu.sync_copy(sel_hbm.at[pl.ds(base, SELW)], selv)

    def grp(g, carry):
        sv = plsc.load_gather(selv, [iota + 16 * g])
        p = (base + 16 * g) + iota
        gid = sv + (p // 50) * 100
        plsc.store_scatter(gidv, [iota + 16 * g], gid)
        return carry

    lax.fori_loop(0, SELW // 16, grp, 0)
    c1 = pltpu.async_copy(acc_hbm.at[gidv], rowa, s1)
    c2 = pltpu.async_copy(xc_hbm.at[gidv], rowb, s2)
    c1.wait()
    c2.wait()

    def row(r, carry):
        rv = jnp.full((16,), r, _i32)
        for j in range(8):
            cv = iota + 16 * j
            a = plsc.load_gather(rowa, [rv, cv])
            b = plsc.load_gather(rowb, [rv, cv])
            plsc.store_scatter(rowa, [rv, cv], a + b)
        return carry

    lax.fori_loop(0, SELW, row, 0)
    pltpu.sync_copy(rowa, out_hbm.at[pl.ds(base, SELW)])


def _final_body(xg_ref, g_ref, bl_ref, w1_ref, b1_ref, w2_ref, b2_ref, o_ref):
    xg = xg_ref[...]
    mu = jnp.mean(xg, axis=-1, keepdims=True)
    var = jnp.mean((xg - mu) ** 2, axis=-1, keepdims=True)
    h = (xg - mu) * lax.rsqrt(var + 1e-5) * g_ref[...] + bl_ref[...]
    t = jnp.maximum(jnp.dot(h, w1_ref[...], preferred_element_type=_f32)
                    + b1_ref[...], 0.0)
    o_ref[...] = jnp.dot(t, w2_ref[...], preferred_element_type=_f32) \
        + b2_ref[...] + xg


def _final_mlp(xg, g, bl, w1, b1, w2, b2):
    fb = 640
    grid = SELPAD // fb
    row = pl.BlockSpec((fb, D), lambda i: (i, 0))
    full = lambda s: pl.BlockSpec(s, lambda i: (0, 0))
    return pl.pallas_call(
        _final_body,
        grid=(grid,),
        in_specs=[row, full((1, D)), full((1, D)), full((D, D)),
                  full((1, D)), full((D, D)), full((1, D))],
        out_specs=row,
        out_shape=jax.ShapeDtypeStruct((SELPAD, D), _f32),
    )(xg, g.reshape(1, D), bl.reshape(1, D), w1, b1.reshape(1, D), w2,
      b2.reshape(1, D))


def kernel(x, edge_index, selected_idx, edge_label, ln1_g, ln1_b, ln2_g,
           ln2_b, c0_W1, c0_b1, c0_W2, c0_b2, c1_W1, c1_b1, c1_W2, c1_b2,
           c2_W1, c2_b1, c2_W2, c2_b2, f_W1, f_b1, f_W2, f_b2):
    src = edge_index[0]
    dst = edge_index[1]
    xp = jnp.pad(x, ((0, NPAD - N), (0, 0)))
    zres = jnp.zeros((NPAD, D), _f32)
    zacc = jnp.zeros((R + 1, D), _f32)
    selp = jnp.pad(selected_idx.reshape(-1), (0, SELPAD - SEL))

    acc, res = xp, zres
    for (w1, b1, w2, b2) in ((c0_W1, c0_b1, c0_W2, c0_b2),
                             (c1_W1, c1_b1, c1_W2, c1_b2),
                             (c2_W1, c2_b1, c2_W2, c2_b2)):
        xcur, u, v = _dense_uv(acc, res, ln1_g, ln1_b, w1, b1)
        pre = _sc_gather_pre(u, v, dst, src)
        m = _edge_mlp(pre, w2, b2)
        acc = _sc_scatter_max(m, dst, zacc)
        res = xcur

    xg = _sc_gather_sel(acc, res, selp)
    out = _final_mlp(xg, ln2_g, ln2_b, f_W1, f_b1, f_W2, f_b2)
    return out[:SEL].reshape(B, K, D)


# speedup vs baseline: 1.2292x; 1.2292x over previous
import functools

import jax
import jax.numpy as jnp
from jax import lax
from jax.experimental import pallas as pl
from jax.experimental.pallas import tpu as pltpu
from jax.experimental.pallas import tpu_sc as plsc

N = 10000
E = 320000
D = 128
B = 100
K = 50

NW = 32
NPAD = 10240
R = NPAD // NW

EW = E // NW
CG = 200
NCHG = EW // CG
NPAIR = NCHG // 2

CS = 8000
NCHS = E // CS
GB = 64
CSQ = CS // 4
QCAP = CSQ + GB

SEL = B * K
SELPAD = 5120
SELW = SELPAD // NW

_f32 = jnp.float32
_i32 = jnp.int32

_mesh = plsc.VectorSubcoreMesh(core_axis_name="c", subcore_axis_name="s")


def _wid():
    return lax.axis_index("s") * 2 + lax.axis_index("c")


def _iota16():
    return lax.broadcasted_iota(_i32, (16,), 0)


def _dense_uv_body(acc_ref, res_ref, g_ref, bl_ref, w1_ref, b1_ref,
                   xcur_ref, u_ref, v_ref):
    xc = acc_ref[...] + res_ref[...]
    xcur_ref[...] = xc
    mu = jnp.mean(xc, axis=-1, keepdims=True)
    var = jnp.mean((xc - mu) ** 2, axis=-1, keepdims=True)
    h = (xc - mu) * lax.rsqrt(var + 1e-5) * g_ref[...] + bl_ref[...]
    w1 = w1_ref[...]
    wv = w1[D:]
    wu = w1[:D] - wv
    u_ref[...] = jnp.dot(h, wu, preferred_element_type=_f32) + b1_ref[...]
    v_ref[...] = jnp.dot(h, wv, preferred_element_type=_f32)


def _dense_uv(acc, res, g, bl, w1, b1):
    nb = 1280
    grid = NPAD // nb
    row = pl.BlockSpec((nb, D), lambda i: (i, 0))
    full = lambda s: pl.BlockSpec(s, lambda i: (0, 0))
    return pl.pallas_call(
        _dense_uv_body,
        grid=(grid,),
        in_specs=[row, row, full((1, D)), full((1, D)), full((2 * D, D)),
                  full((1, D))],
        out_specs=[row, row, row],
        out_shape=[jax.ShapeDtypeStruct((NPAD, D), _f32)] * 3,
    )(acc, res, g.reshape(1, D), bl.reshape(1, D), w1, b1.reshape(1, D))


@functools.partial(
    pl.kernel,
    out_type=jax.ShapeDtypeStruct((E, D), _f32),
    mesh=_mesh,
    compiler_params=pltpu.CompilerParams(needs_layout_passes=False),
    scratch_types=[
        pltpu.VMEM((EW,), _i32),
        pltpu.VMEM((EW,), _i32),
        pltpu.VMEM((CG, D), _f32),
        pltpu.VMEM((CG, D), _f32),
        pltpu.VMEM((CG, D), _f32),
        pltpu.VMEM((CG, D), _f32),
        pltpu.SemaphoreType.DMA,
        pltpu.SemaphoreType.DMA,
        pltpu.SemaphoreType.DMA,
        pltpu.SemaphoreType.DMA,
        pltpu.SemaphoreType.DMA,
        pltpu.SemaphoreType.DMA,
    ],
)
def _sc_gather_pre(u_hbm, v_hbm, dst_hbm, src_hbm, pre_hbm,
                   dsti, srci, bufua, bufva, bufub, bufvb,
                   sua, sva, subb, svb, soa, sob):
    ebase = _wid() * EW
    pltpu.sync_copy(dst_hbm.at[pl.ds(ebase, EW)], dsti)
    pltpu.sync_copy(src_hbm.at[pl.ds(ebase, EW)], srci)

    def fire(c, bufu, bufv, su, sv):
        pltpu.async_copy(u_hbm.at[dsti.at[pl.ds(c * CG, CG)]], bufu, su)
        pltpu.async_copy(v_hbm.at[srci.at[pl.ds(c * CG, CG)]], bufv, sv)

    def wait_g(c, bufu, bufv, su, sv):
        pltpu.make_async_copy(u_hbm.at[dsti.at[pl.ds(c * CG, CG)]],
                              bufu, su).wait()
        pltpu.make_async_copy(v_hbm.at[srci.at[pl.ds(c * CG, CG)]],
                              bufv, sv).wait()

    def compute(bufu, bufv):
        @plsc.parallel_loop(0, CG, step=1, unroll=8)
        def _row(r):
            for j in range(8):
                sl = pl.ds(16 * j, 16)
                bufu[r, sl] = bufu[r, sl] + bufv[r, sl]

    def out_start(c, bufu, so):
        pltpu.async_copy(bufu, pre_hbm.at[pl.ds(ebase + c * CG, CG)], so)

    def out_wait(c, bufu, so):
        pltpu.make_async_copy(bufu, pre_hbm.at[pl.ds(ebase + c * CG, CG)],
                              so).wait()

    fire(0, bufua, bufva, sua, sva)

    def pair(t, carry):
        ca = 2 * t
        cb = ca + 1

        @pl.when(t > 0)
        def _():
            out_wait(cb - 2, bufub, sob)

        fire(cb, bufub, bufvb, subb, svb)
        wait_g(ca, bufua, bufva, sua, sva)
        compute(bufua, bufva)
        out_start(ca, bufua, soa)
        wait_g(cb, bufub, bufvb, subb, svb)
        compute(bufub, bufvb)
        out_start(cb, bufub, sob)

        @pl.when(t < NPAIR - 1)
        def _():
            out_wait(ca, bufua, soa)
            fire(ca + 2, bufua, bufva, sua, sva)

        return carry

    lax.fori_loop(0, NPAIR, pair, 0)
    out_wait(2 * NPAIR - 2, bufua, soa)
    out_wait(2 * NPAIR - 1, bufub, sob)


def _edge_mlp_body(pre_ref, w2_ref, b2_ref, m_ref):
    m_ref[...] = jnp.dot(jnp.maximum(pre_ref[...], 0.0), w2_ref[...],
                         preferred_element_type=_f32) + b2_ref[...]


def _edge_mlp(pre, w2, b2):
    eb = 3200
    grid = E // eb
    row = pl.BlockSpec((eb, D), lambda i: (i, 0))
    full = lambda s: pl.BlockSpec(s, lambda i: (0, 0))
    return pl.pallas_call(
        _edge_mlp_body,
        grid=(grid,),
        in_specs=[row, full((D, D)), full((1, D))],
        out_specs=row,
        out_shape=jax.ShapeDtypeStruct((E, D), _f32),
    )(pre, w2, b2.reshape(1, D))


@functools.partial(
    pl.kernel,
    out_type=jax.ShapeDtypeStruct((NPAD, D), _f32),
    mesh=_mesh,
    compiler_params=pltpu.CompilerParams(needs_layout_passes=False),
    scratch_types=[
        pltpu.VMEM((CS,), _i32),
        pltpu.VMEM((4 * QCAP,), _i32),
        pltpu.VMEM((4 * QCAP,), _i32),
        pltpu.VMEM((R + 1, D), _f32),
        pltpu.VMEM((R + 1, D), _f32),
        pltpu.VMEM((GB, D), _f32),
        pltpu.VMEM((GB, D), _f32),
        pltpu.SemaphoreType.DMA,
        pltpu.SemaphoreType.DMA,
    ],
)
def _sc_scatter_max(m_hbm, dst_hbm, zacc_hbm, acc_hbm,
                    dstv, mids, mdl, acca, accb, mra, mrb, sma, smb):
    wid = _wid()
    lo = wid * R
    iota = _iota16()
    pltpu.sync_copy(zacc_hbm, acca)
    pltpu.sync_copy(zacc_hbm, accb)
    sent_base = wid * GB

    def chunk(c, carry):
        cbase = c * CS
        pltpu.sync_copy(dst_hbm.at[pl.ds(cbase, CS)], dstv)

        def scan(g4, cnt2):
            c2 = cnt2
            base16 = 64 * g4
            for k in range(4):
                off = base16 + 16 * k
                dv = plsc.load_gather(dstv, [iota + off])
                dl = dv - lo
                msk = (dl >= 0) & (dl < R)
                plsc.store_compressed(mids.at[pl.ds(c2, 16)],
                                      iota + (cbase + off), mask=msk)
                plsc.store_compressed(mdl.at[pl.ds(c2, 16)], dl, mask=msk)
                pop = plsc.all_reduce_population_count(msk)
                c2 = c2 + pop[0]
            return c2

        cnt = lax.fori_loop(0, CS // 64, scan, 0)
        for k in range(GB // 16):
            plsc.store_scatter(mids, [cnt + (16 * k) + iota],
                               (sent_base + 16 * k) + iota)
            plsc.store_scatter(mdl, [cnt + (16 * k) + iota],
                               jnp.full((16,), R, _i32))
        numb = (cnt + (GB - 1)) // GB

        def fire(bi, mr, sem):
            pltpu.async_copy(m_hbm.at[mids.at[pl.ds(bi * GB, GB)]], mr, sem)

        def wait(bi, mr, sem):
            pltpu.make_async_copy(m_hbm.at[mids.at[pl.ds(bi * GB, GB)]],
                                  mr, sem).wait()

        def rmw(bi, mr):
            def sub(sg, carry3):
                for i in range(16):
                    p = bi * GB + sg * 16 + i
                    dlv = plsc.load_gather(mdl, [jnp.full((16,), p, _i32)])
                    accx = acca if i % 2 == 0 else accb
                    row = sg * 16 + i
                    mvs = [mr[row, pl.ds(16 * j, 16)] for j in range(8)]
                    olds = [plsc.load_gather(accx, [dlv, iota + 16 * j])
                            for j in range(8)]
                    for j in range(8):
                        plsc.store_scatter(accx, [dlv, iota + 16 * j],
                                           jnp.maximum(olds[j], mvs[j]))
                return carry3

            lax.fori_loop(0, GB // 16, sub, 0)

        @pl.when(numb > 0)
        def _():
            fire(0, mra, sma)

        def batch(bi, carry2):
            even = (bi % 2) == 0
            nxt = bi + 1

            @pl.when((nxt < numb) & even)
            def _():
                fire(nxt, mrb, smb)

            @pl.when((nxt < numb) & jnp.logical_not(even))
            def _():
                fire(nxt, mra, sma)

            @pl.when(even)
            def _():
                wait(bi, mra, sma)
                rmw(bi, mra)

            @pl.when(jnp.logical_not(even))
            def _():
                wait(bi, mrb, smb)
                rmw(bi, mrb)

            return carry2

        lax.fori_loop(0, numb, batch, 0)
        return carry

    lax.fori_loop(0, NCHS, chunk, 0)

    def mrg(r, carry):
        rv = jnp.full((16,), r, _i32)
        for j in range(8):
            cv = iota + 16 * j
            a = plsc.load_gather(acca, [rv, cv])
            b = plsc.load_gather(accb, [rv, cv])
            plsc.store_scatter(acca, [rv, cv], jnp.maximum(a, b))
        return carry

    lax.fori_loop(0, R, mrg, 0)
    pltpu.sync_copy(acca.at[pl.ds(0, R)], acc_hbm.at[pl.ds(lo, R)])


@functools.partial(
    pl.kernel,
    out_type=jax.ShapeDtypeStruct((SELPAD, D), _f32),
    mesh=_mesh,
    compiler_params=pltpu.CompilerParams(needs_layout_passes=False),
    scratch_types=[
        pltpu.VMEM((SELW,), _i32),
        pltpu.VMEM((SELW,), _i32),
        pltpu.VMEM((SELW, D), _f32),
        pltpu.VMEM((SELW, D), _f32),
        pltpu.SemaphoreType.DMA,
        pltpu.SemaphoreType.DMA,
    ],
)
def _sc_gather_sel(acc_hbm, xc_hbm, sel_hbm, out_hbm,
                   selv, gidv, rowa, rowb, s1, s2):
    base = _wid() * SELW
    iota = _iota16()
    pltpu.sync_copy(sel_hbm.at[pl.ds(base, SELW)], selv)

    def grp(g, carry):
        sv = plsc.load_gather(selv, [iota + 16 * g])
        p = (base + 16 * g) + iota
        gid = sv + (p // 50) * 100
        plsc.store_scatter(gidv, [iota + 16 * g], gid)
        return carry

    lax.fori_loop(0, SELW // 16, grp, 0)
    c1 = pltpu.async_copy(acc_hbm.at[gidv], rowa, s1)
    c2 = pltpu.async_copy(xc_hbm.at[gidv], rowb, s2)
    c1.wait()
    c2.wait()

    def row(r, carry):
        rv = jnp.full((16,), r, _i32)
        for j in range(8):
            cv = iota + 16 * j
            a = plsc.load_gather(rowa, [rv, cv])
            b = plsc.load_gather(rowb, [rv, cv])
            plsc.store_scatter(rowa, [rv, cv], a + b)
        return carry

    lax.fori_loop(0, SELW, row, 0)
    pltpu.sync_copy(rowa, out_hbm.at[pl.ds(base, SELW)])


def _final_body(xg_ref, g_ref, bl_ref, w1_ref, b1_ref, w2_ref, b2_ref, o_ref):
    xg = xg_ref[...]
    mu = jnp.mean(xg, axis=-1, keepdims=True)
    var = jnp.mean((xg - mu) ** 2, axis=-1, keepdims=True)
    h = (xg - mu) * lax.rsqrt(var + 1e-5) * g_ref[...] + bl_ref[...]
    t = jnp.maximum(jnp.dot(h, w1_ref[...], preferred_element_type=_f32)
                    + b1_ref[...], 0.0)
    o_ref[...] = jnp.dot(t, w2_ref[...], preferred_element_type=_f32) \
        + b2_ref[...] + xg


def _final_mlp(xg, g, bl, w1, b1, w2, b2):
    fb = 640
    grid = SELPAD // fb
    row = pl.BlockSpec((fb, D), lambda i: (i, 0))
    full = lambda s: pl.BlockSpec(s, lambda i: (0, 0))
    return pl.pallas_call(
        _final_body,
        grid=(grid,),
        in_specs=[row, full((1, D)), full((1, D)), full((D, D)),
                  full((1, D)), full((D, D)), full((1, D))],
        out_specs=row,
        out_shape=jax.ShapeDtypeStruct((SELPAD, D), _f32),
    )(xg, g.reshape(1, D), bl.reshape(1, D), w1, b1.reshape(1, D), w2,
      b2.reshape(1, D))


def kernel(x, edge_index, selected_idx, edge_label, ln1_g, ln1_b, ln2_g,
           ln2_b, c0_W1, c0_b1, c0_W2, c0_b2, c1_W1, c1_b1, c1_W2, c1_b2,
           c2_W1, c2_b1, c2_W2, c2_b2, f_W1, f_b1, f_W2, f_b2):
    src = edge_index[0]
    dst = edge_index[1]
    xp = jnp.pad(x, ((0, NPAD - N), (0, 0)))
    zres = jnp.zeros((NPAD, D), _f32)
    zacc = jnp.zeros((R + 1, D), _f32)
    selp = jnp.pad(selected_idx.reshape(-1), (0, SELPAD - SEL))

    acc, res = xp, zres
    for (w1, b1, w2, b2) in ((c0_W1, c0_b1, c0_W2, c0_b2),
                             (c1_W1, c1_b1, c1_W2, c1_b2),
                             (c2_W1, c2_b1, c2_W2, c2_b2)):
        xcur, u, v = _dense_uv(acc, res, ln1_g, ln1_b, w1, b1)
        pre = _sc_gather_pre(u, v, dst, src)
        m = _edge_mlp(pre, w2, b2)
        acc = _sc_scatter_max(m, dst, zacc)
        res = xcur

    xg = _sc_gather_sel(acc, res, selp)
    out = _final_mlp(xg, ln2_g, ln2_b, f_W1, f_b1, f_W2, f_b2)
    return out[:SEL].reshape(B, K, D)
